# baseline (speedup 1.0000x reference)
import functools

import jax
import jax.numpy as jnp
from jax import lax
from jax.experimental import pallas as pl
from jax.experimental.pallas import tpu as pltpu
from jax.experimental.pallas import tpu_sc as plsc

EPS = 1e-5
BLK = 256


def _sc_row_gather(table, idx, chunk):
    m, = idx.shape
    n, d = table.shape
    info = plsc.get_sparse_core_info()
    nc, ns = info.num_cores, info.num_subcores
    nw = nc * ns
    assert m % (nw * chunk) == 0
    per_w = m // nw
    chunks = per_w // chunk

    mesh = plsc.VectorSubcoreMesh(core_axis_name="c", subcore_axis_name="s")

    @functools.partial(
        pl.kernel,
        mesh=mesh,
        out_type=jax.ShapeDtypeStruct((m, d), table.dtype),
        scratch_types=[
            pltpu.VMEM((chunk,), jnp.int32),
            pltpu.VMEM((chunk,), jnp.int32),
            pltpu.VMEM((chunk, d), table.dtype),
            pltpu.VMEM((chunk, d), table.dtype),
            pltpu.SemaphoreType.DMA,
            pltpu.SemaphoreType.DMA,
            pltpu.SemaphoreType.DMA,
            pltpu.SemaphoreType.DMA,
        ],
    )
    def k(table_hbm, idx_hbm, out_hbm, idx0, idx1, rows0, rows1,
          g0, g1, w0, w1):
        wid = lax.axis_index("s") * nc + lax.axis_index("c")
        base = wid * per_w
        idx_v = [idx0, idx1]
        rows_v = [rows0, rows1]
        gsem = [g0, g1]
        wsem = [w0, w1]
        gcp = [None, None]
        wcp = [None] * chunks

        pltpu.sync_copy(idx_hbm.at[pl.ds(base, chunk)], idx_v[0])
        gcp[0] = pltpu.async_copy(table_hbm.at[idx_v[0]], rows_v[0], gsem[0])
        for i in range(1, chunks):
            b = i % 2
            pltpu.sync_copy(idx_hbm.at[pl.ds(base + i * chunk, chunk)],
                            idx_v[b])
            if i >= 2:
                wcp[i - 2].wait()
            gcp[b] = pltpu.async_copy(table_hbm.at[idx_v[b]], rows_v[b],
                                      gsem[b])
            gcp[1 - b].wait()
            wcp[i - 1] = pltpu.async_copy(
                rows_v[1 - b], out_hbm.at[pl.ds(base + (i - 1) * chunk, chunk)],
                wsem[1 - b])
        last = chunks - 1
        gcp[last % 2].wait()
        wcp[last] = pltpu.async_copy(
            rows_v[last % 2], out_hbm.at[pl.ds(base + last * chunk, chunk)],
            wsem[last % 2])
        if chunks >= 2:
            wcp[last - 1].wait()
        wcp[last].wait()

    return k(table, idx)


def _fused_body(m_ref, xs_ref, w1_ref, b1_ref, g_ref, be_ref, w2_ref, b2_ref,
                ys_ref, hs_s, sum_s, ssq_s, blk):
    ph = pl.program_id(0)
    b = pl.program_id(1)
    e = m_ref[0, b]

    @pl.when(ph == 0)
    def _():
        h = jnp.dot(xs_ref[...].astype(jnp.bfloat16),
                    w1_ref[0].astype(jnp.bfloat16),
                    preferred_element_type=jnp.float32)
        h = h + b1_ref[0, 0][None, :]
        hs_s[pl.ds(b * blk, blk), :] = h
        vc = m_ref[2, b]
        mask = (lax.broadcasted_iota(jnp.int32, h.shape, 0) < vc).astype(
            h.dtype)
        hm = h * mask
        ps = jnp.sum(hm, axis=0, keepdims=True)
        pq = jnp.sum(hm * h, axis=0, keepdims=True)

        @pl.when(m_ref[1, b] == 1)
        def _():
            sum_s[pl.ds(e, 1), :] = ps
            ssq_s[pl.ds(e, 1), :] = pq

        @pl.when(m_ref[1, b] == 0)
        def _():
            sum_s[pl.ds(e, 1), :] += ps
            ssq_s[pl.ds(e, 1), :] += pq

    @pl.when(ph == 1)
    def _():
        cnt = jnp.maximum(m_ref[3, b].astype(jnp.float32), 1.0)
        mean = sum_s[pl.ds(e, 1), :][0] / cnt
        var = ssq_s[pl.ds(e, 1), :][0] / cnt - mean * mean
        rstd = lax.rsqrt(var + EPS)
        scale = rstd * g_ref[0, 0]
        shift = be_ref[0, 0] - mean * scale
        h = hs_s[pl.ds(b * blk, blk), :]
        a = jnp.maximum(h * scale[None, :] + shift[None, :], 0.0)
        y = jnp.dot(a.astype(jnp.bfloat16), w2_ref[0].astype(jnp.bfloat16),
                    preferred_element_type=jnp.float32)
        ys_ref[...] = y + b2_ref[0, 0][None, :]


def kernel(x, attr_idx, W1, b1, gamma, beta, W2, b2):
    bsz, d = x.shape
    e_num, _, h_dim = W1.shape
    o_dim = W2.shape[2]
    nblk = bsz // BLK + e_num
    pad_b = nblk * BLK

    attr = attr_idx.astype(jnp.int32)

    order = jnp.argsort(attr).astype(jnp.int32)
    cnt = jnp.bincount(attr, length=e_num).astype(jnp.int32)
    blocks_e = (cnt + BLK - 1) // BLK
    cumblocks = jnp.cumsum(blocks_e)
    start_block = cumblocks - blocks_e
    blk_ids = jnp.arange(nblk, dtype=jnp.int32)
    eob_raw = jnp.searchsorted(cumblocks, blk_ids, side="right")
    last_e = jnp.max(jnp.where(cnt > 0, jnp.arange(e_num, dtype=jnp.int32), -1))
    eob = jnp.minimum(eob_raw.astype(jnp.int32), last_e)
    r0 = (blk_ids - start_block[eob]) * BLK
    vc = jnp.clip(cnt[eob] - r0, 0, BLK)
    is_first = (blk_ids == start_block[eob]).astype(jnp.int32)

    scnt = jnp.cumsum(cnt) - cnt
    pad_start = start_block * BLK
    p = jnp.arange(pad_b, dtype=jnp.int32)
    pe = eob[p // BLK]
    r = p - pad_start[pe]
    valid = r < cnt[pe]
    gidx = jnp.where(valid, order[jnp.clip(scnt[pe] + r, 0, bsz - 1)],
                     p % bsz)
    gidx = gidx.astype(jnp.int32)

    j = jnp.arange(bsz, dtype=jnp.int32)
    ej = attr[order]
    pos_sorted = pad_start[ej] - scnt[ej] + j
    pos = jnp.zeros(bsz, jnp.int32).at[order].set(pos_sorted)

    meta = jnp.stack([eob, is_first, vc, cnt[eob]])

    xs = _sc_row_gather(x, gidx, chunk=48)

    last_b = nblk - 1
    grid_spec = pltpu.PrefetchScalarGridSpec(
        num_scalar_prefetch=1,
        grid=(2, nblk),
        in_specs=[
            pl.BlockSpec((BLK, d),
                         lambda ph, b, m: (jnp.where(ph == 0, b, last_b), 0)),
            pl.BlockSpec((1, d, h_dim),
                         lambda ph, b, m: (m[0, jnp.where(ph == 0, b, last_b)],
                                           0, 0)),
            pl.BlockSpec((1, 1, h_dim),
                         lambda ph, b, m: (m[0, jnp.where(ph == 0, b, last_b)],
                                           0, 0)),
            pl.BlockSpec((1, 1, h_dim),
                         lambda ph, b, m: (m[0, jnp.where(ph == 0, 0, b)],
                                           0, 0)),
            pl.BlockSpec((1, 1, h_dim),
                         lambda ph, b, m: (m[0, jnp.where(ph == 0, 0, b)],
                                           0, 0)),
            pl.BlockSpec((1, h_dim, o_dim),
                         lambda ph, b, m: (m[0, jnp.where(ph == 0, 0, b)],
                                           0, 0)),
            pl.BlockSpec((1, 1, o_dim),
                         lambda ph, b, m: (m[0, jnp.where(ph == 0, 0, b)],
                                           0, 0)),
        ],
        out_specs=[
            pl.BlockSpec((BLK, o_dim),
                         lambda ph, b, m: (jnp.where(ph == 0, 0, b), 0)),
        ],
        scratch_shapes=[
            pltpu.VMEM((pad_b, h_dim), jnp.float32),
            pltpu.VMEM((e_num, h_dim), jnp.float32),
            pltpu.VMEM((e_num, h_dim), jnp.float32),
        ],
    )
    ys, = pl.pallas_call(
        functools.partial(_fused_body, blk=BLK),
        grid_spec=grid_spec,
        out_shape=[jax.ShapeDtypeStruct((pad_b, o_dim), jnp.float32)],
    )(meta, xs, W1, b1.reshape(e_num, 1, h_dim),
      gamma.reshape(e_num, 1, h_dim), beta.reshape(e_num, 1, h_dim),
      W2, b2.reshape(e_num, 1, o_dim))

    return _sc_row_gather(ys, pos, chunk=32)


# speedup vs baseline: 1.4089x; 1.4089x over previous
import functools

import jax
import jax.numpy as jnp
from jax import lax
from jax.experimental import pallas as pl
from jax.experimental.pallas import tpu as pltpu
from jax.experimental.pallas import tpu_sc as plsc

EPS = 1e-5
BLK = 256


def _sc_row_gather(table, idx, chunk):
    m, = idx.shape
    n, d = table.shape
    info = plsc.get_sparse_core_info()
    nc, ns = info.num_cores, info.num_subcores
    nw = nc * ns
    assert m % (nw * chunk) == 0
    per_w = m // nw
    chunks = per_w // chunk

    mesh = plsc.VectorSubcoreMesh(core_axis_name="c", subcore_axis_name="s")

    @functools.partial(
        pl.kernel,
        mesh=mesh,
        out_type=jax.ShapeDtypeStruct((m, d), table.dtype),
        scratch_types=[
            pltpu.VMEM((chunk,), jnp.int32),
            pltpu.VMEM((chunk,), jnp.int32),
            pltpu.VMEM((chunk, d), table.dtype),
            pltpu.VMEM((chunk, d), table.dtype),
            pltpu.SemaphoreType.DMA,
            pltpu.SemaphoreType.DMA,
            pltpu.SemaphoreType.DMA,
            pltpu.SemaphoreType.DMA,
        ],
    )
    def k(table_hbm, idx_hbm, out_hbm, idx0, idx1, rows0, rows1,
          g0, g1, w0, w1):
        wid = lax.axis_index("s") * nc + lax.axis_index("c")
        base = wid * per_w
        idx_v = [idx0, idx1]
        rows_v = [rows0, rows1]
        gsem = [g0, g1]
        wsem = [w0, w1]
        gcp = [None, None]
        wcp = [None] * chunks

        pltpu.sync_copy(idx_hbm.at[pl.ds(base, chunk)], idx_v[0])
        gcp[0] = pltpu.async_copy(table_hbm.at[idx_v[0]], rows_v[0], gsem[0])
        for i in range(1, chunks):
            b = i % 2
            pltpu.sync_copy(idx_hbm.at[pl.ds(base + i * chunk, chunk)],
                            idx_v[b])
            if i >= 2:
                wcp[i - 2].wait()
            gcp[b] = pltpu.async_copy(table_hbm.at[idx_v[b]], rows_v[b],
                                      gsem[b])
            gcp[1 - b].wait()
            wcp[i - 1] = pltpu.async_copy(
                rows_v[1 - b], out_hbm.at[pl.ds(base + (i - 1) * chunk, chunk)],
                wsem[1 - b])
        last = chunks - 1
        gcp[last % 2].wait()
        wcp[last] = pltpu.async_copy(
            rows_v[last % 2], out_hbm.at[pl.ds(base + last * chunk, chunk)],
            wsem[last % 2])
        if chunks >= 2:
            wcp[last - 1].wait()
        wcp[last].wait()

    return k(table, idx)


def _fused_body(m_ref, xs_ref, w1_ref, b1_ref, g_ref, be_ref, w2_ref, b2_ref,
                ys_ref, hs_s, sum_s, ssq_s, blk):
    ph = pl.program_id(0)
    b = pl.program_id(1)
    e = m_ref[0, b]

    @pl.when(ph == 0)
    def _():
        h = jnp.dot(xs_ref[...], w1_ref[0],
                    preferred_element_type=jnp.float32)
        h = h + b1_ref[0, 0][None, :]
        hs_s[pl.ds(b * blk, blk), :] = h
        vc = m_ref[2, b]
        mask = (lax.broadcasted_iota(jnp.int32, h.shape, 0) < vc).astype(
            h.dtype)
        hm = h * mask
        ps = jnp.sum(hm, axis=0, keepdims=True)
        pq = jnp.sum(hm * h, axis=0, keepdims=True)

        @pl.when(m_ref[1, b] == 1)
        def _():
            sum_s[pl.ds(e, 1), :] = ps
            ssq_s[pl.ds(e, 1), :] = pq

        @pl.when(m_ref[1, b] == 0)
        def _():
            sum_s[pl.ds(e, 1), :] += ps
            ssq_s[pl.ds(e, 1), :] += pq

    @pl.when(ph == 1)
    def _():
        cnt = jnp.maximum(m_ref[3, b].astype(jnp.float32), 1.0)
        mean = sum_s[pl.ds(e, 1), :][0] / cnt
        var = ssq_s[pl.ds(e, 1), :][0] / cnt - mean * mean
        rstd = lax.rsqrt(var + EPS)
        scale = rstd * g_ref[0, 0]
        shift = be_ref[0, 0] - mean * scale
        h = hs_s[pl.ds(b * blk, blk), :]
        a = jnp.maximum(h * scale[None, :] + shift[None, :], 0.0)
        y = jnp.dot(a, w2_ref[0], preferred_element_type=jnp.float32)
        ys_ref[...] = y + b2_ref[0, 0][None, :]


def kernel(x, attr_idx, W1, b1, gamma, beta, W2, b2):
    bsz, d = x.shape
    e_num, _, h_dim = W1.shape
    o_dim = W2.shape[2]
    nblk = bsz // BLK + e_num
    pad_b = nblk * BLK

    attr = attr_idx.astype(jnp.int32)

    e_ids = jnp.arange(e_num, dtype=jnp.int32)
    onehot = (attr[:, None] == e_ids[None, :]).astype(jnp.int32)
    cum = jnp.cumsum(onehot, axis=0)
    cnt = cum[-1]
    rank = jnp.sum((cum - 1) * onehot, axis=1)

    blocks_e = (cnt + BLK - 1) // BLK
    cumblocks = jnp.cumsum(blocks_e)
    start_block = cumblocks - blocks_e
    pad_start = start_block * BLK
    pos = jnp.sum(pad_start[None, :] * onehot, axis=1) + rank

    blk_ids = jnp.arange(nblk, dtype=jnp.int32)
    last_e = jnp.max(jnp.where(cnt > 0, e_ids, -1))
    eob = jnp.minimum(
        jnp.sum((cumblocks[None, :] <= blk_ids[:, None]).astype(jnp.int32),
                axis=1), last_e)
    blk_onehot = (e_ids[None, :] == eob[:, None]).astype(jnp.int32)
    cnt_b = jnp.sum(cnt[None, :] * blk_onehot, axis=1)
    sb_b = jnp.sum(start_block[None, :] * blk_onehot, axis=1)
    r0 = (blk_ids - sb_b) * BLK
    vc = jnp.clip(cnt_b - r0, 0, BLK)
    is_first = (blk_ids == sb_b).astype(jnp.int32)

    p = jnp.arange(pad_b, dtype=jnp.int32)
    gidx = (p % bsz).at[pos].set(jnp.arange(bsz, dtype=jnp.int32))

    meta = jnp.stack([eob, is_first, vc, cnt_b])

    xs = _sc_row_gather(x, gidx, chunk=48)

    last_b = nblk - 1
    grid_spec = pltpu.PrefetchScalarGridSpec(
        num_scalar_prefetch=1,
        grid=(2, nblk),
        in_specs=[
            pl.BlockSpec((BLK, d),
                         lambda ph, b, m: (jnp.where(ph == 0, b, last_b), 0)),
            pl.BlockSpec((1, d, h_dim),
                         lambda ph, b, m: (m[0, jnp.where(ph == 0, b, last_b)],
                                           0, 0)),
            pl.BlockSpec((1, 1, h_dim),
                         lambda ph, b, m: (m[0, jnp.where(ph == 0, b, last_b)],
                                           0, 0)),
            pl.BlockSpec((1, 1, h_dim),
                         lambda ph, b, m: (m[0, jnp.where(ph == 0, 0, b)],
                                           0, 0)),
            pl.BlockSpec((1, 1, h_dim),
                         lambda ph, b, m: (m[0, jnp.where(ph == 0, 0, b)],
                                           0, 0)),
            pl.BlockSpec((1, h_dim, o_dim),
                         lambda ph, b, m: (m[0, jnp.where(ph == 0, 0, b)],
                                           0, 0)),
            pl.BlockSpec((1, 1, o_dim),
                         lambda ph, b, m: (m[0, jnp.where(ph == 0, 0, b)],
                                           0, 0)),
        ],
        out_specs=[
            pl.BlockSpec((BLK, o_dim),
                         lambda ph, b, m: (jnp.where(ph == 0, 0, b), 0)),
        ],
        scratch_shapes=[
            pltpu.VMEM((pad_b, h_dim), jnp.float32),
            pltpu.VMEM((e_num, h_dim), jnp.float32),
            pltpu.VMEM((e_num, h_dim), jnp.float32),
        ],
    )
    ys, = pl.pallas_call(
        functools.partial(_fused_body, blk=BLK),
        grid_spec=grid_spec,
        out_shape=[jax.ShapeDtypeStruct((pad_b, o_dim), jnp.float32)],
    )(meta, xs, W1, b1.reshape(e_num, 1, h_dim),
      gamma.reshape(e_num, 1, h_dim), beta.reshape(e_num, 1, h_dim),
      W2, b2.reshape(e_num, 1, o_dim))

    return _sc_row_gather(ys, pos, chunk=32)
